# Initial kernel scaffold; baseline (speedup 1.0000x reference)
import jax
import jax.numpy as jnp
from jax.experimental import pallas as pl


def kernel(edge_index, emb):
    raise NotImplementedError("write your pallas kernel here")


# speedup vs baseline: 12.0609x; 12.0609x over previous
import jax
import jax.numpy as jnp
from jax import lax
from jax.experimental import pallas as pl
from jax.experimental.pallas import tpu as pltpu
from jax.experimental.pallas import tpu_sc as plsc

N_USERS = 50000
N_NODES = 100000
HALF = 16
LAYERS = 3
N_EDGES = 1600000

N_PAD = 100352
RT = N_PAD // 16
RC = 112
NCH_R = RT // RC

EC = 512
TCH = N_EDGES // EC
CPT = TCH // 16
EXTRA = TCH - CPT * 16


def _rsqrt16(d):
    i = lax.bitcast_convert_type(d, jnp.int32)
    i = jnp.int32(0x5F3759DF) - (i >> 1)
    y = lax.bitcast_convert_type(i, jnp.float32)
    h = -0.5 * d
    for _ in range(3):
        y = y * (1.5 + h * y * y)
    return jnp.where(d > 0.0, y, 0.0)


def _body(row2d, col2d, emb_pair,
          out_pair, y_a, y_b, x1, x2,
          acc, dcis_sp, dris_sp,
          colbuf, rowbuf, gbuf, ones,
          embbuf, aggbuf, ybuf, x1buf, x2buf,
          dcb, drb, sem):
    c = lax.axis_index("c")
    s = lax.axis_index("s")
    rbase = s * RT

    def ione(i, _):
        ones[pl.ds(i * 16, 16)] = jnp.full((16,), 1.0, jnp.float32)
        return 0
    lax.fori_loop(0, 8, ione, 0)

    def izb(i, _):
        aggbuf[i, :] = jnp.zeros((HALF,), jnp.float32)
        return 0
    lax.fori_loop(0, RC, izb, 0)

    def zdc(i, _):
        dcb[pl.ds(i * 16, 16)] = jnp.zeros((16,), jnp.float32)
        return 0
    lax.fori_loop(0, RC // 16, zdc, 0)

    def zdeg(i, _):
        pltpu.sync_copy(dcb, dcis_sp.at[pl.ds(rbase + i * RC, RC)])
        pltpu.sync_copy(dcb, dris_sp.at[pl.ds(rbase + i * RC, RC)])
        return 0
    lax.fori_loop(0, NCH_R, zdeg, 0)
    plsc.subcore_barrier()

    nch = jnp.where(s < EXTRA, CPT + 1, CPT)
    cb = s * CPT + jnp.minimum(s, EXTRA)

    def dgb(i, _):
        g = cb + i
        pltpu.sync_copy(col2d.at[pl.ds(g * 4, 4)], colbuf)
        pltpu.sync_copy(row2d.at[0, pl.ds(g * 4, 4)], rowbuf)
        for j in range(4):
            pltpu.sync_copy(ones, dcis_sp.at[colbuf.at[j]], add=True)
            pltpu.sync_copy(ones, dris_sp.at[rowbuf.at[j]], add=True)
        return 0
    lax.fori_loop(0, nch, dgb, 0)
    plsc.subcore_barrier()

    for dsp in (dcis_sp, dris_sp):
        def inv(i, _):
            rb = rbase + i * RC
            pltpu.sync_copy(dsp.at[pl.ds(rb, RC)], dcb)

            def invc(k, _):
                d = dcb[pl.ds(k * 16, 16)]
                dcb[pl.ds(k * 16, 16)] = _rsqrt16(d)
                return 0
            lax.fori_loop(0, RC // 16, invc, 0)
            pltpu.sync_copy(dcb, dsp.at[pl.ds(rb, RC)])
            return 0
        lax.fori_loop(0, NCH_R, inv, 0)
    plsc.subcore_barrier()

    def y0c(i, _):
        rb = rbase + i * RC
        pltpu.sync_copy(emb_pair.at[c, pl.ds(rb, RC)], embbuf)
        pltpu.sync_copy(dris_sp.at[pl.ds(rb, RC)], drb)

        def rowf(r, _):
            sp = plsc.load_gather(drb, [jnp.zeros((16,), jnp.int32) + r])
            ybuf[r, :] = embbuf[r, :] * sp
            return 0
        lax.fori_loop(0, RC, rowf, 0)
        pltpu.sync_copy(ybuf, y_a.at[pl.ds(c * N_PAD + rb, RC)])
        return 0
    lax.fori_loop(0, NCH_R, y0c, 0)
    plsc.subcore_barrier()

    for l in range(LAYERS):
        ysrc = (y_a, y_b, y_a)[l]
        ydst = (y_b, y_a, None)[l]
        xdst = (x1, x2, None)[l]

        def zc(i, _):
            pltpu.sync_copy(aggbuf, acc.at[pl.ds(rbase + i * RC, RC)])
            return 0
        lax.fori_loop(0, NCH_R, zc, 0)
        plsc.subcore_barrier()

        def sct(i, _):
            g = cb + i
            pltpu.sync_copy(col2d.at[pl.ds(g * 4, 4)], colbuf)
            pltpu.sync_copy(row2d.at[c, pl.ds(g * 4, 4)], rowbuf)
            for j in range(4):
                pltpu.async_copy(ysrc.at[rowbuf.at[j]], gbuf, sem).wait()
                pltpu.sync_copy(gbuf, acc.at[colbuf.at[j]], add=True)
            return 0
        lax.fori_loop(0, nch, sct, 0)
        plsc.subcore_barrier()

        if l < LAYERS - 1:
            def op(i, _):
                rb = rbase + i * RC
                pltpu.sync_copy(acc.at[pl.ds(rb, RC)], aggbuf)
                pltpu.sync_copy(dcis_sp.at[pl.ds(rb, RC)], dcb)
                pltpu.sync_copy(dris_sp.at[pl.ds(rb, RC)], drb)

                def rowf(r, _):
                    idx = jnp.zeros((16,), jnp.int32) + r
                    dc = plsc.load_gather(dcb, [idx])
                    dr = plsc.load_gather(drb, [idx])
                    x = aggbuf[r, :] * dc
                    aggbuf[r, :] = x
                    ybuf[r, :] = x * dr
                    return 0
                lax.fori_loop(0, RC, rowf, 0)
                pltpu.sync_copy(aggbuf, xdst.at[c, pl.ds(rb, RC)])
                pltpu.sync_copy(ybuf, ydst.at[pl.ds(c * N_PAD + rb, RC)])
                return 0
            lax.fori_loop(0, NCH_R, op, 0)
        else:
            def opf(i, _):
                rb = rbase + i * RC
                pltpu.sync_copy(acc.at[pl.ds(rb, RC)], aggbuf)
                pltpu.sync_copy(dcis_sp.at[pl.ds(rb, RC)], dcb)
                pltpu.sync_copy(emb_pair.at[c, pl.ds(rb, RC)], embbuf)
                pltpu.sync_copy(x1.at[c, pl.ds(rb, RC)], x1buf)
                pltpu.sync_copy(x2.at[c, pl.ds(rb, RC)], x2buf)

                def rowf(r, _):
                    idx = jnp.zeros((16,), jnp.int32) + r
                    dc = plsc.load_gather(dcb, [idx])
                    x3 = aggbuf[r, :] * dc
                    m = 0.25 * (embbuf[r, :] + x1buf[r, :] + x2buf[r, :]
                                + x3)
                    aggbuf[r, :] = m
                    return 0
                lax.fori_loop(0, RC, rowf, 0)
                pltpu.sync_copy(aggbuf, out_pair.at[c, pl.ds(rb, RC)])
                return 0
            lax.fori_loop(0, NCH_R, opf, 0)

        plsc.subcore_barrier()

        if l < LAYERS - 1:
            def rz(i, _):
                aggbuf[i, :] = jnp.zeros((HALF,), jnp.float32)
                return 0
            lax.fori_loop(0, RC, rz, 0)


@jax.jit
def _run(row2d, col2d, emb_pair):
    f32 = jnp.float32
    mesh = plsc.VectorSubcoreMesh(core_axis_name="c", subcore_axis_name="s")
    call = pl.kernel(
        _body,
        out_type=[
            jax.ShapeDtypeStruct((2, N_PAD, HALF), f32),
            jax.ShapeDtypeStruct((2 * N_PAD, HALF), f32),
            jax.ShapeDtypeStruct((2 * N_PAD, HALF), f32),
            jax.ShapeDtypeStruct((2, N_PAD, HALF), f32),
            jax.ShapeDtypeStruct((2, N_PAD, HALF), f32),
        ],
        mesh=mesh,
        compiler_params=pltpu.CompilerParams(needs_layout_passes=False,
                                             use_tc_tiling_on_sc=False),
        scratch_types=[
            pltpu.VMEM_SHARED((N_PAD, HALF), f32),
            pltpu.VMEM_SHARED((N_PAD,), f32),
            pltpu.VMEM_SHARED((N_PAD,), f32),
            pltpu.VMEM((4, 128), jnp.int32),
            pltpu.VMEM((4, 128), jnp.int32),
            pltpu.VMEM((128, HALF), f32),
            pltpu.VMEM((128,), f32),
            pltpu.VMEM((RC, HALF), f32),
            pltpu.VMEM((RC, HALF), f32),
            pltpu.VMEM((RC, HALF), f32),
            pltpu.VMEM((RC, HALF), f32),
            pltpu.VMEM((RC, HALF), f32),
            pltpu.VMEM((RC,), f32),
            pltpu.VMEM((RC,), f32),
            pltpu.SemaphoreType.DMA,
        ],
    )
    return call(row2d, col2d, emb_pair)


def kernel(edge_index, emb):
    row = edge_index[0].astype(jnp.int32)
    col = edge_index[1].astype(jnp.int32)
    row2d = jnp.stack([row, row + N_PAD]).reshape(2, N_EDGES // 128, 128)
    col2d = col.reshape(N_EDGES // 128, 128)
    embp = jnp.pad(emb, ((0, N_PAD - N_NODES), (0, 0)))
    emb_pair = jnp.stack([embp[:, :HALF], embp[:, HALF:]])
    out_pair = _run(row2d, col2d, emb_pair)[0]
    full = jnp.concatenate([out_pair[0, :N_NODES], out_pair[1, :N_NODES]],
                           axis=1)
    return full[:N_USERS], full[N_USERS:]
